# baseline (speedup 1.0000x reference)
import functools
import jax
import jax.numpy as jnp
from jax import lax
from jax.experimental import pallas as pl
from jax.experimental.pallas import tpu as pltpu
from jax.experimental.pallas import tpu_sc as plsc

D = 64
NC, NS = 2, 16
NW = NC * NS
BB = 128


def _make_gather(B, L):
    assert B == NW * BB
    n_pairs = L // 2

    @functools.partial(
        pl.kernel,
        mesh=plsc.VectorSubcoreMesh(core_axis_name="c", subcore_axis_name="s"),
        out_type=jax.ShapeDtypeStruct((L, 8, NW, 8, 128), jnp.float32),
        scratch_types=[
            pltpu.VMEM((L, BB), jnp.int32),
            pltpu.VMEM((BB, D), jnp.float32),
            pltpu.VMEM((BB, D), jnp.float32),
            pltpu.VMEM((8, 8, 128), jnp.float32),
            pltpu.VMEM((8, 8, 128), jnp.float32),
            pltpu.SemaphoreType.DMA,
            pltpu.SemaphoreType.DMA,
            pltpu.SemaphoreType.DMA,
            pltpu.SemaphoreType.DMA,
        ],
        compiler_params=pltpu.CompilerParams(
            use_tc_tiling_on_sc=False, needs_layout_passes=False),
    )
    def gather_kernel(idxt_hbm, table_hbm, out_hbm, idx_v,
                      rows0, rows1, stage0, stage1, sg0, sg1, so0, so1):
        rows = [rows0, rows1]
        stage = [stage0, stage1]
        sg = [sg0, sg1]
        so = [so0, so1]
        wid = lax.axis_index("s") * NC + lax.axis_index("c")

        pltpu.sync_copy(idxt_hbm.at[:, pl.ds(wid * BB, BB)], idx_v)

        iota = lax.iota(jnp.int32, 16)
        bvecs = [iota + (16 * b) for b in range(8)]

        def fire_gather(l, p):
            pltpu.async_copy(table_hbm.at[idx_v.at[l]], rows[p], sg[p])

        def drain_gather(p):
            pltpu.make_async_copy(
                table_hbm.at[pl.ds(0, BB)], rows[p], sg[p]
            ).wait()

        def transpose(p):
            for e in range(D):
                es = jnp.full((16,), e, jnp.int32)
                for b in range(8):
                    v = plsc.load_gather(rows[p], [bvecs[b], es])
                    stage[p][e // 8, e % 8, pl.ds(b * 16, 16)] = v

        def fire_out(l, p):
            pltpu.async_copy(stage[p], out_hbm.at[l, :, wid], so[p])

        def drain_out(p):
            pltpu.make_async_copy(
                stage[p], out_hbm.at[0, :, 0], so[p]
            ).wait()

        fire_gather(0, 0)

        def body(m, carry):
            l = 2 * m
            fire_gather(l + 1, 1)
            drain_gather(0)

            @pl.when(m > 0)
            def _():
                drain_out(0)

            transpose(0)
            fire_out(l, 0)

            @pl.when(m < n_pairs - 1)
            def _():
                fire_gather(l + 2, 0)

            drain_gather(1)

            @pl.when(m > 0)
            def _():
                drain_out(1)

            transpose(1)
            fire_out(l + 1, 1)
            return carry

        lax.fori_loop(0, n_pairs, body, 0)
        drain_out(0)
        drain_out(1)

    return gather_kernel


_gather = _make_gather(4096, 200)


def kernel(batch_data, table):
    idxt = batch_data.T.astype(jnp.int32)
    out5d = _gather(idxt, table)
    out = out5d.transpose((2, 4, 0, 1, 3)).reshape(
        batch_data.shape + (D,))
    return out


# speedup vs baseline: 1.3076x; 1.3076x over previous
import functools
import jax
import jax.numpy as jnp
from jax import lax
from jax.experimental import pallas as pl
from jax.experimental.pallas import tpu as pltpu
from jax.experimental.pallas import tpu_sc as plsc

D = 64
NC, NS = 2, 16
NW = NC * NS
BB = 128


def _make_gather(B, L):
    assert B == NW * BB
    n_pairs = L // 2

    @functools.partial(
        pl.kernel,
        mesh=plsc.VectorSubcoreMesh(core_axis_name="c", subcore_axis_name="s"),
        out_type=jax.ShapeDtypeStruct((L, NW, BB, D), jnp.float32),
        scratch_types=[
            pltpu.VMEM((L, BB), jnp.int32),
            pltpu.VMEM((BB, D), jnp.float32),
            pltpu.VMEM((BB, D), jnp.float32),
            pltpu.SemaphoreType.DMA,
            pltpu.SemaphoreType.DMA,
            pltpu.SemaphoreType.DMA,
            pltpu.SemaphoreType.DMA,
        ],
        compiler_params=pltpu.CompilerParams(
            use_tc_tiling_on_sc=False, needs_layout_passes=False),
    )
    def gather_kernel(idxt_hbm, table_hbm, out_hbm, idx_v,
                      rows0, rows1, sg0, sg1, so0, so1):
        rows = [rows0, rows1]
        sg = [sg0, sg1]
        so = [so0, so1]
        wid = lax.axis_index("s") * NC + lax.axis_index("c")

        pltpu.sync_copy(idxt_hbm.at[:, pl.ds(wid * BB, BB)], idx_v)

        def fire_gather(l, p):
            pltpu.async_copy(table_hbm.at[idx_v.at[l]], rows[p], sg[p])

        def drain_gather(p):
            pltpu.make_async_copy(
                table_hbm.at[pl.ds(0, BB)], rows[p], sg[p]
            ).wait()

        def fire_out(l, p):
            pltpu.async_copy(rows[p], out_hbm.at[l, wid], so[p])

        def drain_out(p):
            pltpu.make_async_copy(
                rows[p], out_hbm.at[0, 0], so[p]
            ).wait()

        fire_gather(0, 0)

        def body(m, carry):
            l = 2 * m
            fire_gather(l + 1, 1)
            drain_gather(0)

            @pl.when(m > 0)
            def _():
                drain_out(0)

            fire_out(l, 0)

            @pl.when(m < n_pairs - 1)
            def _():
                fire_gather(l + 2, 0)

            drain_gather(1)

            @pl.when(m > 0)
            def _():
                drain_out(1)

            fire_out(l + 1, 1)
            return carry

        lax.fori_loop(0, n_pairs, body, 0)
        drain_out(0)
        drain_out(1)

    return gather_kernel


_gather = _make_gather(4096, 200)


def _tc_body(in_ref, out_ref):
    x = in_ref[...]
    for u in range(8):
        xu = x[64 * u:64 * u + 64, :]
        xr = xu.reshape(128, 64)
        y = xr.T
        out_ref[0, :, u, :, :] = y.reshape(8, 8, 128)


def _tc_relayout(x2d, L):
    return pl.pallas_call(
        _tc_body,
        grid=(L, 4),
        in_specs=[pl.BlockSpec((512, 128),
                               lambda l, t: (l * 4 + t, 0))],
        out_specs=pl.BlockSpec((1, 8, 8, 8, 128),
                               lambda l, t: (l, 0, t, 0, 0)),
        out_shape=jax.ShapeDtypeStruct((L, 8, NW, 8, 128), jnp.float32),
    )(x2d)


def kernel(batch_data, table):
    B, L = batch_data.shape
    idxt = batch_data.T.astype(jnp.int32)
    mid = _gather(idxt, table)
    mid2d = mid.reshape(L * NW * BB * D // 128, 128)
    out5d = _tc_relayout(mid2d, L)
    out = out5d.transpose((2, 4, 0, 1, 3)).reshape(B, L, D)
    return out


# speedup vs baseline: 1.4127x; 1.0803x over previous
import functools
import jax
import jax.numpy as jnp
from jax import lax
from jax.experimental import pallas as pl
from jax.experimental.pallas import tpu as pltpu
from jax.experimental.pallas import tpu_sc as plsc

D = 64
NC, NS = 2, 16
NW = NC * NS
BB = 128


def _make_gather(B, L):
    assert B == NW * BB
    n_pairs = L // 2

    @functools.partial(
        pl.kernel,
        mesh=plsc.VectorSubcoreMesh(core_axis_name="c", subcore_axis_name="s"),
        out_type=jax.ShapeDtypeStruct((L, NW, BB, D), jnp.float32),
        scratch_types=[
            pltpu.VMEM((L, BB), jnp.int32),
            pltpu.VMEM((BB, D), jnp.float32),
            pltpu.VMEM((BB, D), jnp.float32),
            pltpu.SemaphoreType.DMA,
            pltpu.SemaphoreType.DMA,
            pltpu.SemaphoreType.DMA,
            pltpu.SemaphoreType.DMA,
        ],
        compiler_params=pltpu.CompilerParams(
            use_tc_tiling_on_sc=False, needs_layout_passes=False),
    )
    def gather_kernel(idxt_hbm, table_hbm, out_hbm, idx_v,
                      rows0, rows1, sg0, sg1, so0, so1):
        rows = [rows0, rows1]
        sg = [sg0, sg1]
        so = [so0, so1]
        wid = lax.axis_index("s") * NC + lax.axis_index("c")

        pltpu.sync_copy(idxt_hbm.at[:, pl.ds(wid * BB, BB)], idx_v)

        def fire_gather(l, p):
            pltpu.async_copy(table_hbm.at[idx_v.at[l]], rows[p], sg[p])

        def drain_gather(p):
            pltpu.make_async_copy(
                table_hbm.at[pl.ds(0, BB)], rows[p], sg[p]
            ).wait()

        def fire_out(l, p):
            pltpu.async_copy(rows[p], out_hbm.at[l, wid], so[p])

        def drain_out(p):
            pltpu.make_async_copy(
                rows[p], out_hbm.at[0, 0], so[p]
            ).wait()

        fire_gather(0, 0)

        def body(m, carry):
            l = 2 * m
            fire_gather(l + 1, 1)
            drain_gather(0)

            @pl.when(m > 0)
            def _():
                drain_out(0)

            fire_out(l, 0)

            @pl.when(m < n_pairs - 1)
            def _():
                fire_gather(l + 2, 0)

            drain_gather(1)

            @pl.when(m > 0)
            def _():
                drain_out(1)

            fire_out(l + 1, 1)
            return carry

        lax.fori_loop(0, n_pairs, body, 0)
        drain_out(0)
        drain_out(1)

    return gather_kernel


_gather = _make_gather(4096, 200)


def _tc_body(in_ref, out_ref):
    x = in_ref[...]
    f32 = jnp.float32
    eye = jnp.eye(64, dtype=f32)
    cols = lax.broadcasted_iota(jnp.int32, (64, 128), 1)
    rows = lax.broadcasted_iota(jnp.int32, (64, 128), 0)
    pe = (cols == 2 * rows).astype(f32)
    po = (cols == 2 * rows + 1).astype(f32)
    dn = (((0,), (0,)), ((), ()))
    for u in range(8):
        xu = x[64 * u:64 * u + 64, :]
        a = xu[:, :64]
        b = xu[:, 64:]
        ya = lax.dot_general(a, eye, dn, preferred_element_type=f32)
        yb = lax.dot_general(b, eye, dn, preferred_element_type=f32)
        mm = (((1,), (0,)), ((), ()))
        y = (lax.dot_general(ya, pe, mm, preferred_element_type=f32) +
             lax.dot_general(yb, po, mm, preferred_element_type=f32))
        out_ref[0, :, u, :, :] = y.reshape(8, 8, 128)


def _tc_relayout(x2d, L):
    return pl.pallas_call(
        _tc_body,
        grid=(L, 4),
        in_specs=[pl.BlockSpec((512, 128),
                               lambda l, t: (l * 4 + t, 0))],
        out_specs=pl.BlockSpec((1, 8, 8, 8, 128),
                               lambda l, t: (l, 0, t, 0, 0)),
        out_shape=jax.ShapeDtypeStruct((L, 8, NW, 8, 128), jnp.float32),
    )(x2d)


def kernel(batch_data, table):
    B, L = batch_data.shape
    idxt = batch_data.T.astype(jnp.int32)
    mid = _gather(idxt, table)
    mid2d = mid.reshape(L * NW * BB * D // 128, 128)
    out5d = _tc_relayout(mid2d, L)
    out = out5d.transpose((2, 4, 0, 1, 3)).reshape(B, L, D)
    return out


# speedup vs baseline: 1.6191x; 1.1462x over previous
import functools
import jax
import jax.numpy as jnp
from jax import lax
from jax.experimental import pallas as pl
from jax.experimental.pallas import tpu as pltpu
from jax.experimental.pallas import tpu_sc as plsc

D = 64
NC, NS = 2, 16
NW = NC * NS
CB = 4
SPLITS = ((0, 104), (104, 96))


def _make_gather(B, L):
    per_w = B // NW
    n_chunks = per_w // CB
    n_pairs = n_chunks // 2
    chunk_rows = CB * L

    @functools.partial(
        pl.kernel,
        mesh=plsc.VectorSubcoreMesh(core_axis_name="c", subcore_axis_name="s"),
        out_type=jax.ShapeDtypeStruct((B, L, D), jnp.float32),
        scratch_types=[
            pltpu.VMEM((2, CB, L), jnp.int32),
            pltpu.VMEM((CB, L, D), jnp.float32),
            pltpu.VMEM((CB, L, D), jnp.float32),
            pltpu.SemaphoreType.DMA,
            pltpu.SemaphoreType.DMA,
            pltpu.SemaphoreType.DMA,
            pltpu.SemaphoreType.DMA,
        ],
        compiler_params=pltpu.CompilerParams(use_tc_tiling_on_sc=False),
    )
    def gather_kernel(idx_hbm, table_hbm, out_hbm, idx_v,
                      rows0, rows1, sg0, sg1, so0, so1):
        rows = [rows0, rows1]
        sg = [sg0, sg1]
        so = [so0, so1]
        wid = lax.axis_index("s") * NC + lax.axis_index("c")
        batch0 = wid * per_w

        def fire_gather(ci, p):
            b0 = batch0 + ci * CB
            pltpu.sync_copy(idx_hbm.at[pl.ds(b0, CB)], idx_v.at[p])
            for r in range(CB):
                for off, ln in SPLITS:
                    pltpu.async_copy(
                        table_hbm.at[idx_v.at[p, r, pl.ds(off, ln)]],
                        rows[p].at[r, pl.ds(off, ln)],
                        sg[p],
                    )

        def drain_gather(p):
            pltpu.make_async_copy(
                out_hbm.at[pl.ds(0, CB)], rows[p], sg[p]
            ).wait()

        def fire_out(ci, p):
            b0 = batch0 + ci * CB
            pltpu.async_copy(rows[p], out_hbm.at[pl.ds(b0, CB)], so[p])

        def drain_out(p):
            pltpu.make_async_copy(
                rows[p], out_hbm.at[pl.ds(0, CB)], so[p]
            ).wait()

        fire_gather(0, 0)

        def body(m, carry):
            ci = 2 * m

            @pl.when(m > 0)
            def _():
                drain_out(1)

            fire_gather(ci + 1, 1)
            drain_gather(0)
            fire_out(ci, 0)
            drain_out(0)

            @pl.when(m < n_pairs - 1)
            def _():
                fire_gather(ci + 2, 0)

            drain_gather(1)
            fire_out(ci + 1, 1)
            return carry

        lax.fori_loop(0, n_pairs, body, 0)
        drain_out(1)

    return gather_kernel


_gather = _make_gather(4096, 200)


def kernel(batch_data, table):
    return _gather(batch_data.astype(jnp.int32), table)
